# Initial kernel scaffold; baseline (speedup 1.0000x reference)
import jax
import jax.numpy as jnp
from jax.experimental import pallas as pl


def kernel(x, edge_index, graph_input, batch, params):
    raise NotImplementedError("write your pallas kernel here")


# speedup vs baseline: 2.8041x; 2.8041x over previous
import functools
import math

import jax
import jax.numpy as jnp
from jax.experimental import pallas as pl

_N = 10000
_G = 128
_EPS = 1e-5
_BNS = 1.0 / math.sqrt(1.0 + _EPS)


def _dense_kern(act, x_ref, w_ref, b_ref, a_ref, o_ref):
    z = jax.lax.dot_general(x_ref[...], w_ref[...], (((1,), (0,)), ((), ())),
                            preferred_element_type=jnp.float32)
    z = z + b_ref[...]
    if act:
        z = jnp.where(z >= 0, z, 0.01 * z)
    o_ref[...] = z * a_ref[0:1, :] + a_ref[1:2, :]


def _dense(x, wT, b, scale, shift, act, blk=512):
    rows, fin = x.shape
    fout = wT.shape[1]
    rp = -rows % blk
    xp = jnp.pad(x, ((0, rp), (0, 0)))
    aff = jnp.stack([scale, shift])
    out = pl.pallas_call(
        functools.partial(_dense_kern, act),
        grid=((rows + rp) // blk,),
        in_specs=[pl.BlockSpec((blk, fin), lambda i: (i, 0)),
                  pl.BlockSpec((fin, fout), lambda i: (0, 0)),
                  pl.BlockSpec((1, fout), lambda i: (0, 0)),
                  pl.BlockSpec((2, fout), lambda i: (0, 0))],
        out_specs=pl.BlockSpec((blk, fout), lambda i: (i, 0)),
        out_shape=jax.ShapeDtypeStruct((rows + rp, fout), jnp.float32),
    )(xp, wT, b.reshape(1, -1), aff)
    return out[:rows]


def _mlp3_kern(x_ref, w1, a1, w2, a2, w3, a3, o_ref):
    h = x_ref[...]
    for w, a in ((w1, a1), (w2, a2), (w3, a3)):
        z = jax.lax.dot_general(h, w[...], (((1,), (0,)), ((), ())),
                                preferred_element_type=jnp.float32)
        z = z + a[0:1, :]
        z = jnp.where(z >= 0, z, 0.01 * z)
        h = z * a[1:2, :] + a[2:3, :]
    o_ref[...] = h


def _mlp3(p, t, blk=512):
    rows, fin = t.shape
    fout = p["W3"].shape[0]
    fmid = p["W1"].shape[0]
    rp = -rows % blk
    tp = jnp.pad(t, ((0, rp), (0, 0)))
    a1 = jnp.stack([p["b1"], p["g1"] * _BNS, p["be1"]])
    a2 = jnp.stack([p["b2"], p["g2"] * _BNS, p["be2"]])
    a3 = jnp.stack([p["b3"], p["g3"] * _BNS, p["be3"]])
    out = pl.pallas_call(
        _mlp3_kern,
        grid=((rows + rp) // blk,),
        in_specs=[pl.BlockSpec((blk, fin), lambda i: (i, 0)),
                  pl.BlockSpec((fin, fmid), lambda i: (0, 0)),
                  pl.BlockSpec((3, fmid), lambda i: (0, 0)),
                  pl.BlockSpec((fmid, fout), lambda i: (0, 0)),
                  pl.BlockSpec((3, fout), lambda i: (0, 0)),
                  pl.BlockSpec((fout, fout), lambda i: (0, 0)),
                  pl.BlockSpec((3, fout), lambda i: (0, 0))],
        out_specs=pl.BlockSpec((blk, fout), lambda i: (i, 0)),
        out_shape=jax.ShapeDtypeStruct((rows + rp, fout), jnp.float32),
    )(tp, p["W1"].T, a1, p["W2"].T, a2, p["W3"].T, a3)
    return out[:rows]


def _knn_kern(k, bs, x_ref, sqr_ref, sqc_ref, br_ref, bc_ref, o_ref):
    i = pl.program_id(0)
    n = x_ref.shape[0]
    xb = x_ref[pl.ds(i * bs, bs), :]
    sqb = sqc_ref[pl.ds(i * bs, bs), :]
    bb = bc_ref[pl.ds(i * bs, bs), :]
    dist = sqb + sqr_ref[...] - 2.0 * jax.lax.dot_general(
        xb, x_ref[...], (((1,), (1,)), ((), ())),
        preferred_element_type=jnp.float32)
    row_ids = jax.lax.broadcasted_iota(jnp.int32, (bs, n), 0) + i * bs
    col_ids = jax.lax.broadcasted_iota(jnp.int32, (bs, n), 1)
    valid = (bb == br_ref[...]) & (row_ids != col_ids)
    dist = jnp.where(valid, dist, jnp.float32(1e30))
    cols = []
    for _ in range(k):
        m = jnp.min(dist, axis=1, keepdims=True)
        cand = jnp.where(dist == m, col_ids, jnp.int32(2 ** 30))
        am = jnp.min(cand, axis=1, keepdims=True)
        cols.append(am)
        dist = jnp.where(col_ids == am, jnp.float32(jnp.inf), dist)
    o_ref[...] = jnp.concatenate(cols, axis=1)


def _knn(x, batch, k, bs=256):
    n, f = x.shape
    npad = -n % bs
    npt = n + npad
    xp = jnp.pad(x, ((0, npad), (0, 0)))
    bp = jnp.pad(batch, (0, npad), constant_values=-1)
    sq = jnp.sum(xp * xp, axis=1)
    out = pl.pallas_call(
        functools.partial(_knn_kern, k, bs),
        grid=(npt // bs,),
        in_specs=[pl.BlockSpec((npt, f), lambda i: (0, 0)),
                  pl.BlockSpec((1, npt), lambda i: (0, 0)),
                  pl.BlockSpec((npt, 1), lambda i: (0, 0)),
                  pl.BlockSpec((1, npt), lambda i: (0, 0)),
                  pl.BlockSpec((npt, 1), lambda i: (0, 0))],
        out_specs=pl.BlockSpec((bs, k), lambda i: (i, 0)),
        out_shape=jax.ShapeDtypeStruct((npt, k), jnp.int32),
    )(xp, sq.reshape(1, npt), sq.reshape(npt, 1),
      bp.reshape(1, npt), bp.reshape(npt, 1))
    return out[:n]


def _shortcut(p, x):
    return _dense(x, p["Ws"].T, p["bs"], p["gs"] * _BNS, p["bes"], act=False)


def _edge_conv1(p, x, src, dst):
    x_i = x[dst]
    x_j = x[src]
    msg = _mlp3(p, jnp.concatenate([x_i, x_j - x_i], axis=1))
    agg = jax.ops.segment_sum(msg, dst, num_segments=_N)
    cnt = jax.ops.segment_sum(jnp.ones(dst.shape, jnp.float32), dst,
                              num_segments=_N)
    agg = agg / jnp.maximum(cnt, 1.0)[:, None]
    return agg + _shortcut(p, x)


def _edge_conv_knn(p, x, nbr):
    n, k = nbr.shape
    f = x.shape[1]
    x_j = x[nbr.reshape(-1)]
    x_i = jnp.broadcast_to(x[:, None, :], (n, k, f)).reshape(n * k, f)
    msg = _mlp3(p, jnp.concatenate([x_i, x_j - x_i], axis=1))
    agg = jnp.mean(msg.reshape(n, k, -1), axis=1)
    return agg + _shortcut(p, x)


def kernel(x, edge_index, graph_input, batch, params):
    cnt = jax.ops.segment_sum(jnp.ones((_N,), jnp.float32), batch,
                              num_segments=_G)
    cnt = jnp.maximum(cnt, 1.0)
    mean = jax.ops.segment_sum(x, batch, num_segments=_G) / cnt[:, None]
    h0 = x - mean[batch] * params["gn_ms"]
    var = jax.ops.segment_sum(h0 * h0, batch, num_segments=_G) / cnt[:, None]
    h0 = params["gn_w"] * h0 / jnp.sqrt(var + _EPS)[batch] + params["gn_b"]

    c1 = _edge_conv1(params["conv1"], h0, edge_index[0], edge_index[1])
    c2 = _edge_conv_knn(params["conv2"], c1, _knn(c1, batch, 4))
    c3 = _edge_conv_knn(params["conv3"], c2, _knn(c2, batch, 4))
    c4 = _edge_conv_knn(params["conv4"], c3, _knn(c3, batch, 3))

    h = jnp.concatenate([c1, c2, c3, c4], axis=1)

    pp = params["pool"]
    one32 = jnp.ones((32,), jnp.float32)
    a1 = _dense(h, pp["Wa1"].T, pp["ba1"], one32, jnp.zeros((32,), jnp.float32),
                act=True)
    wa2 = jnp.pad(pp["Wa2"], ((0, 7), (0, 0)))
    ba2 = jnp.pad(pp["ba2"], (0, 7))
    one8 = jnp.ones((8,), jnp.float32)
    a = _dense(a1, wa2.T, ba2, one8, jnp.zeros((8,), jnp.float32),
               act=False)[:, 0]
    amax = jax.ops.segment_max(a, batch, num_segments=_G)
    ex = jnp.exp(a - amax[batch])
    denom = jax.ops.segment_sum(ex, batch, num_segments=_G)
    w = ex / denom[batch]
    g = jax.ops.segment_sum(h * w[:, None], batch, num_segments=_G)

    g = jnp.concatenate([g, graph_input], axis=1)
    hp = params["head"]
    g = hp["g0"] * g * _BNS + hp["be0"]
    g = _dense(g, hp["Wd1"].T, hp["bd1"], hp["g1"] * _BNS, hp["be1"], act=True)
    g = _dense(g, hp["Wd2"].T, hp["bd2"], hp["g2"] * _BNS, hp["be2"], act=True)
    wo = jnp.pad(hp["Wo"], ((0, 4), (0, 0)))
    bo = jnp.pad(hp["bo"], (0, 4))
    one8f = jnp.ones((8,), jnp.float32)
    out = _dense(g, wo.T, bo, one8f, jnp.zeros((8,), jnp.float32), act=False)
    return out[:, :4]
